# baseline (speedup 1.0000x reference)
import functools

import jax
import jax.numpy as jnp
from jax.experimental import pallas as pl
from jax.experimental.pallas import tpu as pltpu

N_CLS = 1000
N_CTX = 4
CTX_DIM = 512
CTX_LEN = 77
SUFFIX_LEN = CTX_LEN - 1 - N_CTX

CB = 25


def _body(prefix_ref, ctx_ref, suffix_ref, out_ref):
    out_ref[:, 0:1, :] = prefix_ref[...]
    out_ref[:, 1:1 + N_CTX, :] = jnp.broadcast_to(
        ctx_ref[...][None], (CB, N_CTX, CTX_DIM))
    out_ref[:, 1 + N_CTX:, :] = suffix_ref[...]


def kernel(prefixs, ctx, suffixs):
    grid = (N_CLS // CB,)
    return pl.pallas_call(
        _body,
        grid=grid,
        in_specs=[
            pl.BlockSpec((CB, 1, CTX_DIM), lambda i: (i, 0, 0)),
            pl.BlockSpec((N_CTX, CTX_DIM), lambda i: (0, 0)),
            pl.BlockSpec((CB, SUFFIX_LEN, CTX_DIM), lambda i: (i, 0, 0)),
        ],
        out_specs=pl.BlockSpec((CB, CTX_LEN, CTX_DIM), lambda i: (i, 0, 0)),
        out_shape=jax.ShapeDtypeStruct((N_CLS, CTX_LEN, CTX_DIM), jnp.float32),
        compiler_params=pltpu.CompilerParams(
            dimension_semantics=("arbitrary",),
        ),
    )(prefixs, ctx, suffixs)
